# baseline (speedup 1.0000x reference)
import jax
import jax.numpy as jnp
from jax.experimental import pallas as pl


_ROWS_PER_BLOCK = 4096


def _one_hot_body(idx_ref, out_ref):
    iota = jax.lax.broadcasted_iota(jnp.int32, (_ROWS_PER_BLOCK, 25), 1)
    out_ref[:] = (idx_ref[:] == iota).astype(jnp.float32)


def kernel(input):
    idx = input.astype(jnp.int32).reshape(-1, 1)
    n = idx.shape[0]
    grid = n // _ROWS_PER_BLOCK
    return pl.pallas_call(
        _one_hot_body,
        grid=(grid,),
        in_specs=[pl.BlockSpec((_ROWS_PER_BLOCK, 1), lambda i: (i, 0))],
        out_specs=pl.BlockSpec((_ROWS_PER_BLOCK, 25), lambda i: (i, 0)),
        out_shape=jax.ShapeDtypeStruct((n, 25), jnp.float32),
    )(idx)


# speedup vs baseline: 1.1152x; 1.1152x over previous
import functools

import jax
import jax.numpy as jnp
from jax import lax
from jax.experimental import pallas as pl
from jax.experimental.pallas import tpu as pltpu
from jax.experimental.pallas import tpu_sc as plsc

_NC = 2
_NS = 16
_NW = _NC * _NS
_L = 16

_N_ROWS = 16384 * 100
_K = 25
_C = 800
_CHUNK_OUT = _C * _K
_ROWS_PER_W = _N_ROWS // _NW
_CHUNKS_PER_W = _ROWS_PER_W // _C
_OUT_PER_W = _ROWS_PER_W * _K


def _scatter_val(out_buf, idx_buf, val_vec):
    iota25 = lax.iota(jnp.int32, _L) * _K

    def body(i, _):
        iv = idx_buf[pl.ds(i * _L, _L)]
        pos = i * (_L * _K) + iota25 + iv
        plsc.store_scatter(out_buf, [pos], val_vec)
        return 0

    lax.fori_loop(0, _C // _L, body, 0)


def _one_hot_sc(idx_hbm, out_hbm, ob0, ob1, ib0, ib1, ib2, ib3,
                osem0, osem1, isem0, isem1, isem2, isem3):
    obufs = [ob0, ob1]
    ibufs = [ib0, ib1, ib2, ib3]
    osems = [osem0, osem1]
    isems = [isem0, isem1, isem2, isem3]

    wid = lax.axis_index("s") * _NC + lax.axis_index("c")
    ibase = wid * _ROWS_PER_W
    obase = wid * _OUT_PER_W

    ones = jnp.full((_L,), 1.0, jnp.float32)
    zeros = jnp.zeros((_L,), jnp.float32)

    def zbody(i, _):
        ob0[pl.ds(i * _L, _L)] = zeros
        ob1[pl.ds(i * _L, _L)] = zeros
        return 0

    lax.fori_loop(0, _CHUNK_OUT // _L, zbody, 0)

    def idx_dma(g, b):
        return pltpu.make_async_copy(
            idx_hbm.at[pl.ds(ibase + g * _C, _C)], ibufs[b], isems[b])

    def out_dma(g, b):
        return pltpu.make_async_copy(
            obufs[b], out_hbm.at[pl.ds(obase + g * _CHUNK_OUT, _CHUNK_OUT)],
            osems[b])

    idx_dma(0, 0).start()
    idx_dma(1, 1).start()

    def outer(o, _):
        for b in range(4):
            g = o * 4 + b
            b2 = b % 2
            bprev = (b + 2) % 4

            idx_dma(g, b).wait()

            @pl.when(g >= 2)
            def _():
                out_dma(g - 2, b2).wait()
                _scatter_val(obufs[b2], ibufs[bprev], zeros)

            _scatter_val(obufs[b2], ibufs[b], ones)
            out_dma(g, b2).start()

            @pl.when(g + 2 < _CHUNKS_PER_W)
            def _():
                idx_dma(g + 2, bprev).start()
        return 0

    lax.fori_loop(0, _CHUNKS_PER_W // 4, outer, 0)

    out_dma(_CHUNKS_PER_W - 2, 0).wait()
    out_dma(_CHUNKS_PER_W - 1, 1).wait()


@functools.partial(jax.jit, static_argnums=())
def _run(idx_flat):
    mesh = plsc.VectorSubcoreMesh(core_axis_name="c", subcore_axis_name="s")
    fn = pl.kernel(
        _one_hot_sc,
        out_type=jax.ShapeDtypeStruct((_N_ROWS * _K,), jnp.float32),
        mesh=mesh,
        compiler_params=pltpu.CompilerParams(needs_layout_passes=False),
        scratch_types=[
            pltpu.VMEM((_CHUNK_OUT,), jnp.float32),
            pltpu.VMEM((_CHUNK_OUT,), jnp.float32),
            pltpu.VMEM((_C,), jnp.int32),
            pltpu.VMEM((_C,), jnp.int32),
            pltpu.VMEM((_C,), jnp.int32),
            pltpu.VMEM((_C,), jnp.int32),
            pltpu.SemaphoreType.DMA,
            pltpu.SemaphoreType.DMA,
            pltpu.SemaphoreType.DMA,
            pltpu.SemaphoreType.DMA,
            pltpu.SemaphoreType.DMA,
            pltpu.SemaphoreType.DMA,
        ],
    )
    return fn(idx_flat)


def kernel(input):
    idx_flat = input.astype(jnp.int32).reshape(-1)
    out_flat = _run(idx_flat)
    return out_flat.reshape(_N_ROWS, _K)


# speedup vs baseline: 8.0409x; 7.2106x over previous
import jax
import jax.numpy as jnp
from jax import lax
from jax.experimental import pallas as pl

_N = 16384 * 100
_K = 25
_R = 8192


def _body(idx_ref, out_ref):
    iota = lax.broadcasted_iota(jnp.int32, (_K, _R), 0)
    out_ref[:] = (idx_ref[:] == iota).astype(jnp.float32)


def kernel(input):
    idx = input.astype(jnp.int32).reshape(1, _N)
    out_t = pl.pallas_call(
        _body,
        grid=(_N // _R,),
        in_specs=[pl.BlockSpec((1, _R), lambda i: (0, i))],
        out_specs=pl.BlockSpec((_K, _R), lambda i: (0, i)),
        out_shape=jax.ShapeDtypeStruct((_K, _N), jnp.float32),
    )(idx)
    return out_t.T


# speedup vs baseline: 12.8762x; 1.6013x over previous
import functools

import jax
import jax.numpy as jnp
from jax import lax
from jax.experimental import pallas as pl
from jax.experimental.pallas import tpu as pltpu
from jax.experimental.pallas import tpu_sc as plsc

_NC = 2
_NS = 16
_NW = _NC * _NS
_L = 16

_N = 16384 * 100
_K = 25
_CH = 1280
_COLS_PER_W = _N // _NW
_CHUNKS_PER_W = _COLS_PER_W // _CH


def _scatter_val(out_buf, idx_buf, val_vec):
    iota = lax.iota(jnp.int32, _L)

    def body(i, _):
        iv = idx_buf[pl.ds(i * _L, _L)]
        rvec = i * _L + iota
        plsc.store_scatter(out_buf, [iv, rvec], val_vec)
        return 0

    lax.fori_loop(0, _CH // _L, body, 0)


def _one_hot_sc(idx_hbm, out_hbm, ob0, ob1, ib0, ib1, ib2, ib3,
                osem0, osem1, isem0, isem1, isem2, isem3):
    obufs = [ob0, ob1]
    ibufs = [ib0, ib1, ib2, ib3]
    osems = [osem0, osem1]
    isems = [isem0, isem1, isem2, isem3]

    wid = lax.axis_index("s") * _NC + lax.axis_index("c")
    col0 = wid * _COLS_PER_W

    ones = jnp.full((_L,), 1.0, jnp.float32)
    zeros = jnp.zeros((_L,), jnp.float32)

    def zbody(j, _):
        for c in range(_K):
            ob0[c, pl.ds(j * _L, _L)] = zeros
            ob1[c, pl.ds(j * _L, _L)] = zeros
        return 0

    lax.fori_loop(0, _CH // _L, zbody, 0)

    def idx_dma(g, b):
        return pltpu.make_async_copy(
            idx_hbm.at[pl.ds(col0 + g * _CH, _CH)], ibufs[b], isems[b])

    def out_dma(g, b):
        return pltpu.make_async_copy(
            obufs[b], out_hbm.at[:, pl.ds(col0 + g * _CH, _CH)], osems[b])

    idx_dma(0, 0).start()
    idx_dma(1, 1).start()

    def outer(o, _):
        for b in range(4):
            g = o * 4 + b
            b2 = b % 2
            bprev = (b + 2) % 4

            idx_dma(g, b).wait()

            @pl.when(g >= 2)
            def _():
                out_dma(g - 2, b2).wait()
                _scatter_val(obufs[b2], ibufs[bprev], zeros)

            _scatter_val(obufs[b2], ibufs[b], ones)
            out_dma(g, b2).start()

            @pl.when(g + 2 < _CHUNKS_PER_W)
            def _():
                idx_dma(g + 2, bprev).start()
        return 0

    lax.fori_loop(0, _CHUNKS_PER_W // 4, outer, 0)

    out_dma(_CHUNKS_PER_W - 2, 0).wait()
    out_dma(_CHUNKS_PER_W - 1, 1).wait()


@functools.partial(jax.jit, static_argnums=())
def _run(idx):
    mesh = plsc.VectorSubcoreMesh(core_axis_name="c", subcore_axis_name="s")
    fn = pl.kernel(
        _one_hot_sc,
        out_type=jax.ShapeDtypeStruct((_K, _N), jnp.float32),
        mesh=mesh,
        compiler_params=pltpu.CompilerParams(needs_layout_passes=False),
        scratch_types=[
            pltpu.VMEM((_K, _CH), jnp.float32),
            pltpu.VMEM((_K, _CH), jnp.float32),
            pltpu.VMEM((_CH,), jnp.int32),
            pltpu.VMEM((_CH,), jnp.int32),
            pltpu.VMEM((_CH,), jnp.int32),
            pltpu.VMEM((_CH,), jnp.int32),
            pltpu.SemaphoreType.DMA,
            pltpu.SemaphoreType.DMA,
            pltpu.SemaphoreType.DMA,
            pltpu.SemaphoreType.DMA,
            pltpu.SemaphoreType.DMA,
            pltpu.SemaphoreType.DMA,
        ],
    )
    return fn(idx)


def kernel(input):
    out_t = _run(input.astype(jnp.int32).reshape(-1))
    return out_t.T
